# baseline (speedup 1.0000x reference)
import functools

import jax
import jax.numpy as jnp
from jax import lax
from jax.experimental import pallas as pl
from jax.experimental.pallas import tpu as pltpu
from jax.experimental.pallas import tpu_sc as plsc

N_NODES = 100000
D_FEAT = 128
N_GROUPS = 8
EMB = 16
ENC = 2 * EMB

SPECIAL_ROWS = 8000
CHUNK = 2000
NCHUNK = N_NODES // CHUNK
SPECIAL_CHUNKS = SPECIAL_ROWS // CHUNK
NBUF = 8
OUTLAG = 4
ENC_ROWS = SPECIAL_ROWS + CHUNK

SC_CHUNK = 200
SC_NCHUNK = ENC_ROWS // SC_CHUNK
SC_SPECIAL_CHUNKS = SPECIAL_ROWS // SC_CHUNK
NC, NS = 2, 16
NW = NC * NS


def _sc_lookup(gt_hbm, ht_hbm, enc_hbm, gt_v, ht_v, enc_v):
    wid = lax.axis_index("s") * NC + lax.axis_index("c")
    pltpu.sync_copy(gt_hbm, gt_v)
    pltpu.sync_copy(ht_hbm, ht_v)
    gt0 = gt_v[0, :]
    ht0 = ht_v[0, :]
    ht1 = ht_v[1, :]
    for r in range(SC_CHUNK):
        enc_v[r, 0:EMB] = gt0
        enc_v[r, EMB:ENC] = ht0 if r % 2 == 0 else ht1
    for j in range(-(-SC_NCHUNK // NW)):
        c = wid + NW * j

        @pl.when(c < SC_NCHUNK)
        def _chunk():
            @pl.when(c < SC_SPECIAL_CHUNKS)
            def _patch():
                g = c // 5
                acc = jnp.zeros_like(gt0)
                for gg in range(N_GROUPS):
                    acc = jnp.where(g == gg, gt_v[gg, :], acc)
                enc_v[0, 0:EMB] = acc
                enc_v[100, 0:EMB] = acc

            pltpu.sync_copy(enc_v, enc_hbm.at[pl.ds(c * SC_CHUNK, SC_CHUNK), :])

            @pl.when(c < SC_SPECIAL_CHUNKS)
            def _restore():
                enc_v[0, 0:EMB] = gt0
                enc_v[100, 0:EMB] = gt0


def _sc_encode(group_table, hemi_table):
    mesh = plsc.VectorSubcoreMesh(
        core_axis_name="c", subcore_axis_name="s",
        num_cores=NC, num_subcores=NS)
    f = functools.partial(
        pl.kernel,
        out_type=jax.ShapeDtypeStruct((ENC_ROWS, ENC), jnp.float32),
        mesh=mesh,
        scratch_types=[
            pltpu.VMEM((N_GROUPS, EMB), jnp.float32),
            pltpu.VMEM((2, EMB), jnp.float32),
            pltpu.VMEM((SC_CHUNK, ENC), jnp.float32),
        ],
    )(_sc_lookup)
    return f(group_table, hemi_table)


def _tc_body(x_hbm, enc_hbm, o_hbm, obuf, insem, encsem, outsem):
    def start_in(j):
        s = j % NBUF
        pltpu.make_async_copy(
            x_hbm.at[pl.ds(j * CHUNK, CHUNK), :],
            obuf.at[s].at[:, pl.ds(0, D_FEAT)],
            insem.at[s]).start()
        if j < SPECIAL_CHUNKS:
            pltpu.make_async_copy(
                enc_hbm.at[pl.ds(j * CHUNK, CHUNK), :],
                obuf.at[s].at[:, pl.ds(D_FEAT, ENC)],
                encsem.at[s]).start()
        elif NBUF <= j < NBUF + SPECIAL_CHUNKS:
            pltpu.make_async_copy(
                enc_hbm.at[pl.ds(SPECIAL_ROWS, CHUNK), :],
                obuf.at[s].at[:, pl.ds(D_FEAT, ENC)],
                encsem.at[s]).start()

    def wait_in(j):
        s = j % NBUF
        pltpu.make_async_copy(
            x_hbm.at[pl.ds(j * CHUNK, CHUNK), :],
            obuf.at[s].at[:, pl.ds(0, D_FEAT)],
            insem.at[s]).wait()
        if j < SPECIAL_CHUNKS or NBUF <= j < NBUF + SPECIAL_CHUNKS:
            pltpu.make_async_copy(
                enc_hbm.at[pl.ds(0, CHUNK), :],
                obuf.at[s].at[:, pl.ds(D_FEAT, ENC)],
                encsem.at[s]).wait()

    def start_out(j):
        s = j % NBUF
        pltpu.make_async_copy(
            obuf.at[s], o_hbm.at[pl.ds(j * CHUNK, CHUNK), :],
            outsem.at[s]).start()

    def wait_out(j):
        s = j % NBUF
        pltpu.make_async_copy(
            obuf.at[s], o_hbm.at[pl.ds(j * CHUNK, CHUNK), :],
            outsem.at[s]).wait()

    for s in range(NBUF):
        pltpu.make_async_copy(
            enc_hbm.at[pl.ds(SPECIAL_ROWS, CHUNK), :],
            obuf.at[s].at[:, pl.ds(D_FEAT, ENC)],
            encsem.at[s]).start()
    for s in range(NBUF):
        pltpu.make_async_copy(
            enc_hbm.at[pl.ds(SPECIAL_ROWS, CHUNK), :],
            obuf.at[s].at[:, pl.ds(D_FEAT, ENC)],
            encsem.at[s]).wait()

    for j in range(NBUF):
        start_in(j)

    for k in range(NCHUNK):
        wait_in(k)
        start_out(k)
        r = k - OUTLAG
        if r >= 0:
            wait_out(r)
            if r + NBUF < NCHUNK:
                start_in(r + NBUF)

    for r in range(max(0, NCHUNK - OUTLAG), NCHUNK):
        wait_out(r)


def kernel(x, group_table, hemi_table):
    n = x.shape[0]
    enc = _sc_encode(group_table, hemi_table)
    return pl.pallas_call(
        _tc_body,
        in_specs=[
            pl.BlockSpec(memory_space=pl.ANY),
            pl.BlockSpec(memory_space=pl.ANY),
        ],
        out_specs=pl.BlockSpec(memory_space=pl.ANY),
        out_shape=jax.ShapeDtypeStruct((n, D_FEAT + ENC), jnp.float32),
        scratch_shapes=[
            pltpu.VMEM((NBUF, CHUNK, D_FEAT + ENC), jnp.float32),
            pltpu.SemaphoreType.DMA((NBUF,)),
            pltpu.SemaphoreType.DMA((NBUF,)),
            pltpu.SemaphoreType.DMA((NBUF,)),
        ],
    )(x, enc)


# speedup vs baseline: 1.0222x; 1.0222x over previous
import functools

import jax
import jax.numpy as jnp
from jax import lax
from jax.experimental import pallas as pl
from jax.experimental.pallas import tpu as pltpu
from jax.experimental.pallas import tpu_sc as plsc

N_NODES = 100000
D_FEAT = 128
N_GROUPS = 8
EMB = 16
ENC = 2 * EMB
WIDTH = D_FEAT + ENC

SPECIAL_ROWS = 8000
R = 200
NCH = N_NODES // R
NSPECIAL = SPECIAL_ROWS // R
NC, NS = 2, 16
NW = NC * NS
PER_W = -(-NCH // NW)
NB = 2


def _sc_body(x_hbm, gt_hbm, ht_hbm, o_hbm, gt_v, ht_v, vbuf, xsem, osem):
    wid = lax.axis_index("s") * NC + lax.axis_index("c")
    pltpu.sync_copy(gt_hbm, gt_v)
    pltpu.sync_copy(ht_hbm, ht_v)
    gt0 = gt_v[0, :]
    ht0 = ht_v[0, :]
    ht1 = ht_v[1, :]
    for s in range(NB):
        for r in range(R):
            vbuf[s, r, D_FEAT:D_FEAT + EMB] = gt0
            vbuf[s, r, D_FEAT + EMB:] = ht0 if r % 2 == 0 else ht1

    def chunk_of(t):
        return wid + NW * t

    def start_x(t):
        c = chunk_of(t)
        s = t % NB
        pltpu.make_async_copy(
            x_hbm.at[pl.ds(c * R, R), :],
            vbuf.at[s].at[:, pl.ds(0, D_FEAT)],
            xsem.at[s]).start()

    def wait_x(t):
        s = t % NB
        c = chunk_of(t)
        pltpu.make_async_copy(
            x_hbm.at[pl.ds(c * R, R), :],
            vbuf.at[s].at[:, pl.ds(0, D_FEAT)],
            xsem.at[s]).wait()

    def start_o(t):
        c = chunk_of(t)
        s = t % NB
        pltpu.make_async_copy(
            vbuf.at[s], o_hbm.at[pl.ds(c * R, R), :],
            osem.at[s]).start()

    def wait_o(t):
        c = chunk_of(t)
        s = t % NB
        pltpu.make_async_copy(
            vbuf.at[s], o_hbm.at[pl.ds(c * R, R), :],
            osem.at[s]).wait()

    @pl.when(chunk_of(0) < NCH)
    def _p0():
        start_x(0)

    for t in range(PER_W):
        c = chunk_of(t)

        @pl.when(c < NCH)
        def _iter(t=t, c=c):
            wait_x(t)
            g = c // 5
            acc = gt0
            for gg in range(1, N_GROUPS):
                take = jnp.logical_and(c < NSPECIAL, g == gg)
                acc = jnp.where(take, gt_v[gg, :], acc)
            s = t % NB
            vbuf[s, 0, D_FEAT:D_FEAT + EMB] = acc
            vbuf[s, 100, D_FEAT:D_FEAT + EMB] = acc
            start_o(t)
            if t + 1 < PER_W:
                if t + 1 - NB >= 0:
                    @pl.when(chunk_of(t + 1 - NB) < NCH)
                    def _wo():
                        wait_o(t + 1 - NB)

                @pl.when(chunk_of(t + 1) < NCH)
                def _nx():
                    start_x(t + 1)

    for t in range(max(0, PER_W - NB), PER_W):
        @pl.when(chunk_of(t) < NCH)
        def _drain(t=t):
            wait_o(t)


def kernel(x, group_table, hemi_table):
    n = x.shape[0]
    mesh = plsc.VectorSubcoreMesh(
        core_axis_name="c", subcore_axis_name="s",
        num_cores=NC, num_subcores=NS)
    f = functools.partial(
        pl.kernel,
        out_type=jax.ShapeDtypeStruct((n, WIDTH), jnp.float32),
        mesh=mesh,
        scratch_types=[
            pltpu.VMEM((N_GROUPS, EMB), jnp.float32),
            pltpu.VMEM((2, EMB), jnp.float32),
            pltpu.VMEM((NB, R, WIDTH), jnp.float32),
            pltpu.SemaphoreType.DMA((NB,)),
            pltpu.SemaphoreType.DMA((NB,)),
        ],
    )(_sc_body)
    return f(x, group_table, hemi_table)


# speedup vs baseline: 1.0275x; 1.0051x over previous
import functools

import jax
import jax.numpy as jnp
from jax import lax
from jax.experimental import pallas as pl
from jax.experimental.pallas import tpu as pltpu
from jax.experimental.pallas import tpu_sc as plsc

N_NODES = 100000
D_FEAT = 128
N_GROUPS = 8
EMB = 16
ENC = 2 * EMB
WIDTH = D_FEAT + ENC

SPECIAL_ROWS = 8000
R = 200
NCH = N_NODES // R
NSPECIAL = SPECIAL_ROWS // R
NC, NS = 2, 16
NW = NC * NS
PER_W = -(-NCH // NW)
NB = 3


def _sc_body(x_hbm, gt_hbm, ht_hbm, o_hbm,
             gt_v, ht_v, xbuf, enc_v, spec_v, xsem, asem, bsem):
    wid = lax.axis_index("s") * NC + lax.axis_index("c")
    pltpu.sync_copy(gt_hbm, gt_v)
    pltpu.sync_copy(ht_hbm, ht_v)
    gt0 = gt_v[0, :]
    ht0 = ht_v[0, :]
    ht1 = ht_v[1, :]
    for r in range(R):
        hemi = ht0 if r % 2 == 0 else ht1
        enc_v[r, 0:EMB] = gt0
        enc_v[r, EMB:ENC] = hemi
        spec_v[r, 0:EMB] = gt0
        spec_v[r, EMB:ENC] = hemi

    def chunk_of(t):
        return wid + NW * t

    def start_x(t):
        pltpu.make_async_copy(
            x_hbm.at[pl.ds(chunk_of(t) * R, R), :],
            xbuf.at[t % NB], xsem.at[t % NB]).start()

    def wait_x(t):
        pltpu.make_async_copy(
            x_hbm.at[pl.ds(chunk_of(t) * R, R), :],
            xbuf.at[t % NB], xsem.at[t % NB]).wait()

    def a_copy(t):
        return pltpu.make_async_copy(
            xbuf.at[t % NB],
            o_hbm.at[pl.ds(chunk_of(t) * R, R), pl.ds(0, D_FEAT)],
            asem.at[t % NB])

    def b_copy(t):
        return pltpu.make_async_copy(
            enc_v,
            o_hbm.at[pl.ds(chunk_of(t) * R, R), pl.ds(D_FEAT, ENC)],
            bsem.at[t % NB])

    @pl.when(chunk_of(0) < NCH)
    def _p0():
        start_x(0)

    if PER_W > 1:
        @pl.when(chunk_of(1) < NCH)
        def _p1():
            start_x(1)

    for t in range(PER_W):
        c = chunk_of(t)

        @pl.when(c < NCH)
        def _iter(t=t, c=c):
            wait_x(t)
            a_copy(t).start()

            @pl.when(c < NSPECIAL)
            def _special():
                g = c // 5
                acc = gt0
                for gg in range(1, N_GROUPS):
                    acc = jnp.where(g == gg, gt_v[gg, :], acc)
                spec_v[0, 0:EMB] = acc
                spec_v[100, 0:EMB] = acc
                pltpu.sync_copy(
                    spec_v,
                    o_hbm.at[pl.ds(c * R, R), pl.ds(D_FEAT, ENC)])

            @pl.when(c >= NSPECIAL)
            def _steady():
                b_copy(t).start()

            if t - 1 >= 0:
                a_copy(t - 1).wait()

                @pl.when(chunk_of(t - 1) >= NSPECIAL)
                def _wb():
                    b_copy(t - 1).wait()

            if t + 2 < PER_W:
                @pl.when(chunk_of(t + 2) < NCH)
                def _nx():
                    start_x(t + 2)

    for t in range(PER_W):
        c = chunk_of(t)
        last = jnp.logical_and(c < NCH, chunk_of(t + 1) >= NCH)

        @pl.when(last)
        def _drain(t=t, c=c):
            a_copy(t).wait()

            @pl.when(c >= NSPECIAL)
            def _wb():
                b_copy(t).wait()


def kernel(x, group_table, hemi_table):
    n = x.shape[0]
    mesh = plsc.VectorSubcoreMesh(
        core_axis_name="c", subcore_axis_name="s",
        num_cores=NC, num_subcores=NS)
    f = functools.partial(
        pl.kernel,
        out_type=jax.ShapeDtypeStruct((n, WIDTH), jnp.float32),
        mesh=mesh,
        scratch_types=[
            pltpu.VMEM((N_GROUPS, EMB), jnp.float32),
            pltpu.VMEM((2, EMB), jnp.float32),
            pltpu.VMEM((NB, R, D_FEAT), jnp.float32),
            pltpu.VMEM((R, ENC), jnp.float32),
            pltpu.VMEM((R, ENC), jnp.float32),
            pltpu.SemaphoreType.DMA((NB,)),
            pltpu.SemaphoreType.DMA((NB,)),
            pltpu.SemaphoreType.DMA((NB,)),
        ],
    )(_sc_body)
    return f(x, group_table, hemi_table)


# speedup vs baseline: 1.0279x; 1.0004x over previous
import functools

import jax
import jax.numpy as jnp
from jax import lax
from jax.experimental import pallas as pl
from jax.experimental.pallas import tpu as pltpu
from jax.experimental.pallas import tpu_sc as plsc

N_NODES = 100000
D_FEAT = 128
N_GROUPS = 8
EMB = 16
ENC = 2 * EMB
WIDTH = D_FEAT + ENC

SPECIAL_ROWS = 8000
R = 200
NCH = N_NODES // R
NSPECIAL = SPECIAL_ROWS // R
NC, NS = 2, 16
NW = NC * NS
PER_W = -(-NCH // NW)
NB = 3


def _sc_body(x_hbm, gt_hbm, ht_hbm, o_hbm,
             gt_v, ht_v, xbuf, enc_v, spec_v, xsem, asem, bsem):
    wid = lax.axis_index("s") * NC + lax.axis_index("c")
    pltpu.sync_copy(gt_hbm, gt_v)
    pltpu.sync_copy(ht_hbm, ht_v)
    gt0 = gt_v[0, :]
    ht0 = ht_v[0, :]
    ht1 = ht_v[1, :]
    for r in range(R):
        hemi = ht0 if r % 2 == 0 else ht1
        enc_v[r, 0:EMB] = gt0
        enc_v[r, EMB:ENC] = hemi
        spec_v[r, 0:EMB] = gt0
        spec_v[r, EMB:ENC] = hemi

    def chunk_of(t):
        return wid + NW * t

    def start_x(t):
        pltpu.make_async_copy(
            x_hbm.at[pl.ds(chunk_of(t) * R, R), :],
            xbuf.at[t % NB], xsem.at[t % NB]).start()

    def wait_x(t):
        pltpu.make_async_copy(
            x_hbm.at[pl.ds(chunk_of(t) * R, R), :],
            xbuf.at[t % NB], xsem.at[t % NB]).wait()

    def a_copy(t):
        return pltpu.make_async_copy(
            xbuf.at[t % NB],
            o_hbm.at[pl.ds(chunk_of(t) * R, R), pl.ds(0, D_FEAT)],
            asem.at[t % NB])

    def b_copy(t):
        return pltpu.make_async_copy(
            enc_v,
            o_hbm.at[pl.ds(chunk_of(t) * R, R), pl.ds(D_FEAT, ENC)],
            bsem.at[t % NB])

    @pl.when(chunk_of(0) < NCH)
    def _p0():
        start_x(0)

    if PER_W > 1:
        @pl.when(chunk_of(1) < NCH)
        def _p1():
            start_x(1)

    for t in range(PER_W):
        c = chunk_of(t)

        @pl.when(c < NCH)
        def _iter(t=t, c=c):
            @pl.when(c < NSPECIAL)
            def _special():
                g = c // 5
                acc = gt0
                for gg in range(1, N_GROUPS):
                    acc = jnp.where(g == gg, gt_v[gg, :], acc)
                spec_v[0, 0:EMB] = acc
                spec_v[100, 0:EMB] = acc
                pltpu.sync_copy(
                    spec_v,
                    o_hbm.at[pl.ds(c * R, R), pl.ds(D_FEAT, ENC)])

            @pl.when(c >= NSPECIAL)
            def _steady():
                b_copy(t).start()

            wait_x(t)
            a_copy(t).start()

            if t - 1 >= 0:
                a_copy(t - 1).wait()

                @pl.when(chunk_of(t - 1) >= NSPECIAL)
                def _wb():
                    b_copy(t - 1).wait()

            if t + 2 < PER_W:
                @pl.when(chunk_of(t + 2) < NCH)
                def _nx():
                    start_x(t + 2)

    for t in range(PER_W):
        c = chunk_of(t)
        last = jnp.logical_and(c < NCH, chunk_of(t + 1) >= NCH)

        @pl.when(last)
        def _drain(t=t, c=c):
            a_copy(t).wait()

            @pl.when(c >= NSPECIAL)
            def _wb():
                b_copy(t).wait()


def kernel(x, group_table, hemi_table):
    n = x.shape[0]
    mesh = plsc.VectorSubcoreMesh(
        core_axis_name="c", subcore_axis_name="s",
        num_cores=NC, num_subcores=NS)
    f = functools.partial(
        pl.kernel,
        out_type=jax.ShapeDtypeStruct((n, WIDTH), jnp.float32),
        mesh=mesh,
        scratch_types=[
            pltpu.VMEM((N_GROUPS, EMB), jnp.float32),
            pltpu.VMEM((2, EMB), jnp.float32),
            pltpu.VMEM((NB, R, D_FEAT), jnp.float32),
            pltpu.VMEM((R, ENC), jnp.float32),
            pltpu.VMEM((R, ENC), jnp.float32),
            pltpu.SemaphoreType.DMA((NB,)),
            pltpu.SemaphoreType.DMA((NB,)),
            pltpu.SemaphoreType.DMA((NB,)),
        ],
    )(_sc_body)
    return f(x, group_table, hemi_table)
